# baseline (speedup 1.0000x reference)
import jax
import jax.numpy as jnp
from jax.experimental import pallas as pl
from jax.experimental.pallas import tpu as pltpu

_POS_W = 10.0
_SMOOTH = 1.0
_M_VAR = 1.0
_M_DIST = 5.0
_K_INST = 4
_K_ROWS = 8
_L = 128
_TH = 64
_VMEM = 64 * 1024 * 1024


def _params(*sem):
    return pltpu.CompilerParams(dimension_semantics=tuple(sem),
                                vmem_limit_bytes=_VMEM)


def _sp_neg(x):
    return jnp.maximum(-x, 0.0) + jnp.log(1.0 + jnp.exp(-jnp.abs(x)))


def _k1_body(x_ref, w_ref, bias_ref, gseg_ref, goff_ref, gz_ref, iseg_ref,
             head_ref, b0_ref, i0_ref, t0_ref, ob_ref, zb_ref,
             b2_ref, i2_ref, t2_ref):
    @pl.when(pl.program_id(1) == 0)
    def _init():
        for r in (b0_ref, i0_ref, t0_ref, ob_ref, zb_ref, b2_ref, i2_ref, t2_ref):
            r[...] = jnp.zeros_like(r)

    w = w_ref[...]
    bias = bias_ref[...]
    r0, r3, r4, r5 = [], [], [], []
    for th in range(_TH):
        h = jnp.dot(w, x_ref[:, th, :],
                    preferred_element_type=jnp.float32) + bias
        head_ref[:, th, :] = h
        r0.append(h[0:1])
        r3.append(h[3:4])
        r4.append(h[4:5])
        r5.append(h[5:6])
    x0 = jnp.concatenate(r0, axis=0)
    x3 = jnp.concatenate(r3, axis=0)
    x4 = jnp.concatenate(r4, axis=0)
    x5 = jnp.concatenate(r5, axis=0)

    seg = gseg_ref[0]

    sp0 = _sp_neg(x0)
    sg0 = jnp.exp(-sp0)
    b0_ref[...] += _POS_W * seg * sp0 + (1.0 - seg) * (sp0 + x0)
    i0_ref[...] += sg0 * seg
    t0_ref[...] += sg0 + seg

    spn = _sp_neg(x3)
    fg = seg > 0.5
    nlp = jnp.where(fg, jnp.minimum(spn, 100.0), 100.0)
    nl1 = jnp.where(fg, jnp.minimum(spn + x3, 100.0), 0.0)
    t_off = goff_ref[0]
    ob_ref[...] += t_off * nlp + (1.0 - t_off) * nl1

    err = seg * x4 - gz_ref[0]
    zb_ref[...] += err * err

    t2 = iseg_ref[0]
    sp2 = _sp_neg(x5)
    sg2 = jnp.exp(-sp2)
    b2_ref[...] += _POS_W * t2 * sp2 + (1.0 - t2) * (sp2 + x5)
    i2_ref[...] += sg2 * t2
    t2_ref[...] += sg2 + t2


def _run_k1(x_bchw, w_t, bias, gseg, goff, gz, iseg):
    B, cin, H, W = x_bchw.shape
    nt = H // _TH
    gt_spec = pl.BlockSpec((None, 1, _TH, W), lambda b, t: (b, 0, t, 0))
    acc_spec = pl.BlockSpec((None, _TH, W), lambda b, t: (b, 0, 0))
    outs = pl.pallas_call(
        _k1_body,
        out_shape=(jax.ShapeDtypeStruct((B, 8, H, W), jnp.float32),)
        + (jax.ShapeDtypeStruct((B, _TH, W), jnp.float32),) * 8,
        grid=(B, nt),
        in_specs=[pl.BlockSpec((None, cin, _TH, W), lambda b, t: (b, 0, t, 0)),
                  pl.BlockSpec((8, cin), lambda b, t: (0, 0)),
                  pl.BlockSpec((8, 1), lambda b, t: (0, 0)),
                  gt_spec, gt_spec, gt_spec, gt_spec],
        out_specs=(pl.BlockSpec((None, 8, _TH, W), lambda b, t: (b, 0, t, 0)),)
        + (acc_spec,) * 8,
        compiler_params=_params("parallel", "arbitrary"),
    )(x_bchw, w_t, bias, gseg, goff, gz, iseg)
    return outs[0], outs[1:]


def _inst_masks(inst, P):
    ids_i = jax.lax.broadcasted_iota(jnp.int32, (_K_ROWS, P), 0) + 1
    onehot = jnp.where((inst == ids_i.astype(jnp.float32)) & (ids_i <= _K_INST),
                       1.0, 0.0)
    cnt = jnp.sum(onehot, axis=1, keepdims=True)
    valid = jnp.where(cnt > 0.0, 1.0, 0.0)
    safe = jnp.maximum(cnt, 1.0)
    ones_p = jnp.ones((1, P), jnp.float32)
    cnt_row = jax.lax.dot_general(ones_p, onehot, (((1,), (1,)), ((), ())),
                                  preferred_element_type=jnp.float32)
    valid_row = jnp.where(cnt_row > 0.0, 1.0, 0.0)
    return onehot, cnt, valid, safe, valid_row


def _pushpull(emb, masks):
    onehot, cnt, valid, safe, valid_row = masks
    sums = jax.lax.dot_general(onehot, emb, (((1,), (1,)), ((), ())),
                               preferred_element_type=jnp.float32)
    centers = sums / safe
    e_sq = jnp.sum(emb * emb, axis=0, keepdims=True)
    c_sq = jnp.sum(centers * centers, axis=1, keepdims=True)
    ec = jnp.dot(centers, emb, preferred_element_type=jnp.float32)
    dist = jnp.sqrt(jnp.maximum(e_sq - 2.0 * ec + c_sq, 0.0))
    pull = jnp.sum(valid * onehot * jnp.maximum(dist - _M_VAR, 0.0) / safe)

    cc = jax.lax.dot_general(centers, centers, (((1,), (1,)), ((), ())),
                             preferred_element_type=jnp.float32)
    ri = jax.lax.broadcasted_iota(jnp.int32, (_K_ROWS, _K_ROWS), 0)
    ci = jax.lax.broadcasted_iota(jnp.int32, (_K_ROWS, _K_ROWS), 1)
    eye = jnp.where(ri == ci, 1.0, 0.0)
    diag = jnp.sum(cc * eye, axis=0, keepdims=True)
    cd = jnp.sqrt(jnp.maximum(c_sq + diag - 2.0 * cc, 0.0))
    push = jnp.sum(valid * valid_row * (1.0 - eye)
                   * jnp.maximum(_M_DIST - cd, 0.0))
    return pull, push, jnp.sum(valid)


def _k2a_body(emb_ref, emb2_ref, inst_ref, inst2_ref, *out_refs):
    P = emb_ref.shape[1]
    pu1, ps1, c1 = _pushpull(emb_ref[...], _inst_masks(inst_ref[...], P))
    pu2, ps2, c2 = _pushpull(emb2_ref[...], _inst_masks(inst2_ref[...], P))
    for r, s in zip(out_refs, (pu1, ps1, c1, pu2, ps2, c2)):
        r[...] = jnp.broadcast_to(s, (1, _L))


def _run_k2a(emb, emb2, inst, inst2):
    B, _, P = emb.shape
    hb = B // 2
    bidx = lambda i, j: i * hb + j
    e_spec = pl.BlockSpec((None, 2, P), lambda i, j: (bidx(i, j), 0, 0))
    i_spec = pl.BlockSpec((None, 1, P), lambda i, j: (bidx(i, j), 0, 0))
    o_spec = pl.BlockSpec((None, 1, _L), lambda i, j: (bidx(i, j), 0, 0))
    return pl.pallas_call(
        _k2a_body,
        out_shape=(jax.ShapeDtypeStruct((B, 1, _L), jnp.float32),) * 6,
        grid=(2, hb),
        in_specs=[e_spec, e_spec, i_spec, i_spec],
        out_specs=(o_spec,) * 6,
        compiler_params=_params("parallel", "arbitrary"),
    )(emb, emb2, inst, inst2)


def _k2b_body(wemb_ref, inst2_ref, predw_ref, isegr_ref, *out_refs):
    P = wemb_ref.shape[1]
    pu3, ps3, c3 = _pushpull(wemb_ref[...], _inst_masks(inst2_ref[...], P))

    x = predw_ref[...]
    t = isegr_ref[...]
    sp = _sp_neg(x)
    sg = jnp.exp(-sp)
    bce = jnp.sum(_POS_W * t * sp + (1.0 - t) * (sp + x), axis=0, keepdims=True)
    inter = jnp.sum(sg * t, axis=0, keepdims=True)
    total = jnp.sum(sg + t, axis=0, keepdims=True)

    for r, s in zip(out_refs[:3], (pu3, ps3, c3)):
        r[...] = jnp.broadcast_to(s, (1, _L))
    out_refs[3][...] = bce
    out_refs[4][...] = inter
    out_refs[5][...] = total


def _run_k2b(wemb, inst2, predw, isegr):
    B, _, P = wemb.shape
    R = predw.shape[1]
    hb = B // 2
    bidx = lambda i, j: i * hb + j
    e_spec = pl.BlockSpec((None, 2, P), lambda i, j: (bidx(i, j), 0, 0))
    i_spec = pl.BlockSpec((None, 1, P), lambda i, j: (bidx(i, j), 0, 0))
    r_spec = pl.BlockSpec((None, R, _L), lambda i, j: (bidx(i, j), 0, 0))
    o_spec = pl.BlockSpec((None, 1, _L), lambda i, j: (bidx(i, j), 0, 0))
    return pl.pallas_call(
        _k2b_body,
        out_shape=(jax.ShapeDtypeStruct((B, 1, _L), jnp.float32),) * 6,
        grid=(2, hb),
        in_specs=[e_spec, i_spec, r_spec, r_spec],
        out_specs=(o_spec,) * 6,
        compiler_params=_params("parallel", "arbitrary"),
    )(wemb, inst2, predw, isegr)


def _make_k3_body(n):
    def _k3_body(b0_ref, i0_ref, t0_ref, ob_ref, zb_ref, b2_ref, i2_ref,
                 t2_ref, pp_a_ref, pp_b_ref, seg_h_ref, out_ref):
        _k3_math(n, b0_ref, i0_ref, t0_ref, ob_ref, zb_ref, b2_ref, i2_ref,
                 t2_ref, pp_a_ref, pp_b_ref, seg_h_ref, out_ref)
    return _k3_body


def _k3_math(n, b0_ref, i0_ref, t0_ref, ob_ref, zb_ref, b2_ref, i2_ref,
             t2_ref, pp_a_ref, pp_b_ref, seg_h_ref, out_ref):
    def _seg(bce, inter, total):
        i_s = jnp.sum(inter)
        u_s = jnp.sum(total) - i_s
        return bce / n + (1.0 - (i_s + _SMOOTH) / (u_s + _SMOOTH))

    loss_seg = _seg(jnp.sum(b0_ref[...]), i0_ref[...], t0_ref[...])
    loss_seg_2d = _seg(jnp.sum(b2_ref[...]), i2_ref[...], t2_ref[...])
    loss_offset_m = jnp.sum(ob_ref[...]) / n
    loss_z_m = jnp.sum(zb_ref[...]) / n

    ppa = jnp.sum(pp_a_ref[...], axis=(0, 2)) * (1.0 / _L)
    ppb = jnp.sum(pp_b_ref[...], axis=(0, 2)) * (1.0 / _L)

    def _pp_total(pu, ps, c):
        return (ps + pu) / jnp.maximum(c, 1.0)

    loss_emb = _pp_total(ppa[0], ppa[1], ppa[2])
    loss_emb_2d = _pp_total(ppa[3], ppa[4], ppa[5])
    loss_emb_hg = _pp_total(ppb[0], ppb[1], ppb[2])

    sh = seg_h_ref[...]
    loss_seg_hg = _seg(jnp.sum(sh[:, 0]), sh[:, 1], sh[:, 2])

    vals = (3.0 * loss_seg + 0.5 * loss_emb,
            3.0 * loss_seg_2d + 0.5 * loss_emb_2d,
            60.0 * loss_offset_m,
            30.0 * loss_z_m,
            3.0 * loss_seg_hg + 0.5 * loss_emb_hg,
            loss_seg_hg,
            loss_emb_hg,
            loss_seg_hg)
    out_ref[...] = jnp.concatenate(
        [jnp.broadcast_to(v, (1, _L)) for v in vals], axis=0)


def _run_k3(accs, ppa, ppb, seg_h, n):
    B = ppa.shape[0]
    full = lambda s: pl.BlockSpec(s, lambda: tuple(0 for _ in s))
    in_specs = [full(a.shape) for a in accs] + [
        full((B, 6, _L)), full((B, 3, _L)), full((B, 3, _L))]
    return pl.pallas_call(
        _make_k3_body(n),
        out_shape=jax.ShapeDtypeStruct((8, _L), jnp.float32),
        in_specs=in_specs,
        out_specs=full((8, _L)),
        compiler_params=_params(),
    )(*accs, ppa, ppb, seg_h)


def _inv3(m):
    a, b, c = m[..., 0, 0], m[..., 0, 1], m[..., 0, 2]
    d, e, f = m[..., 1, 0], m[..., 1, 1], m[..., 1, 2]
    g, h, i = m[..., 2, 0], m[..., 2, 1], m[..., 2, 2]
    ca = e * i - f * h
    cb = -(d * i - f * g)
    cc = d * h - e * g
    cd = -(b * i - c * h)
    ce = a * i - c * g
    cf = -(a * h - b * g)
    cg = b * f - c * e
    ch = -(a * f - c * d)
    ci = a * e - b * d
    det = a * ca + b * cb + c * cc
    adj = jnp.stack([jnp.stack([ca, cd, cg], -1),
                     jnp.stack([cb, ce, ch], -1),
                     jnp.stack([cc, cf, ci], -1)], -2)
    return adj / det[..., None, None]


def _norm_pix(h, w):
    wd = float(w - 1) if w != 1 else 1e-14
    hd = float(h - 1) if h != 1 else 1e-14
    return jnp.array([[2.0 / wd, 0.0, -1.0],
                      [0.0, 2.0 / hd, -1.0],
                      [0.0, 0.0, 1.0]], jnp.float32)


def _warp3(src_flat, m_pix, H, W):
    n_t = _norm_pix(H, W)
    m_norm = n_t @ (m_pix @ _inv3(n_t))
    m_s2d = _inv3(m_norm)
    xs = jnp.linspace(-1.0, 1.0, W, dtype=jnp.float32)
    ys = jnp.linspace(-1.0, 1.0, H, dtype=jnp.float32)
    gx, gy = jnp.meshgrid(xs, ys)
    pts = jnp.stack([gx.ravel(), gy.ravel(),
                     jnp.ones(H * W, jnp.float32)], -1)
    tp = pts @ m_s2d.T
    zd = tp[:, 2]
    zd = jnp.where(jnp.abs(zd) > 1e-8, zd, 1e-8)
    sx = (tp[:, 0] / zd + 1.0) * 0.5 * (W - 1)
    sy = (tp[:, 1] / zd + 1.0) * 0.5 * (H - 1)
    x0 = jnp.floor(sx)
    y0 = jnp.floor(sy)
    wx1 = sx - x0
    wy1 = sy - y0

    def tap(ix, iy, wgt):
        ok = ((ix >= 0) & (ix <= W - 1) & (iy >= 0) & (iy <= H - 1))
        idx = (jnp.clip(iy, 0, H - 1) * W + jnp.clip(ix, 0, W - 1)).astype(jnp.int32)
        g = jnp.take(src_flat, idx, axis=1)
        return g * jnp.where(ok, wgt, 0.0)[None, :]

    return (tap(x0, y0, (1.0 - wx1) * (1.0 - wy1))
            + tap(x0 + 1.0, y0, wx1 * (1.0 - wy1))
            + tap(x0, y0 + 1.0, (1.0 - wx1) * wy1)
            + tap(x0 + 1.0, y0 + 1.0, wx1 * wy1))


def kernel(inputs, images_gt, gt_seg, gt_instance, gt_offset_y, gt_z,
           image_gt_segment, image_gt_instance, w_bev, b_bev, w_2d, b_2d,
           homography):
    B, cin, H, W = inputs.shape
    P = H * W
    R = P // _L
    w_t = jnp.transpose(jnp.concatenate([w_bev, w_2d], axis=1))
    bias = jnp.transpose(jnp.concatenate([b_bev, b_2d], axis=1))

    head, accs = _run_k1(inputs, w_t, bias, gt_seg, gt_offset_y, gt_z,
                         image_gt_segment)

    hm = jnp.broadcast_to(homography[None], (B, 3, 3))
    hm_inv = _inv3(hm)
    fro = jnp.sqrt(jnp.sum(hm_inv * hm_inv, axis=(1, 2), keepdims=True))
    hm_inv = hm_inv / jnp.maximum(fro, 1e-6)
    n_t = _norm_pix(H, W)[None]
    hm_inv = _inv3(n_t) @ (hm_inv @ n_t)

    emb = head[:, 1:3].reshape(B, 2, P)
    emb2 = head[:, 6:8].reshape(B, 2, P)
    inst = gt_instance.reshape(B, 1, P)
    inst2 = image_gt_instance.reshape(B, 1, P)
    k2a = _run_k2a(emb, emb2, inst, inst2)

    warp_src = head[:, 5:8].reshape(B * 3, P)
    warped = _warp3(warp_src, hm_inv[0], H, W).reshape(B, 3, P)
    wemb = warped[:, 1:3]
    predw = warped[:, 0].reshape(B, R, _L)
    isegr = image_gt_segment.reshape(B, R, _L)
    k2b = _run_k2b(wemb, inst2, predw, isegr)

    ppa = jnp.concatenate(k2a, axis=1)
    ppb = jnp.concatenate(k2b[:3], axis=1)
    seg_h = jnp.concatenate(k2b[3:], axis=1)
    fin = _run_k3(accs, ppa, ppb, seg_h, float(B * P))

    loss_total = fin[0:1, 0]
    loss_total_2d = fin[1:2, 0]
    loss_offset = fin[2:3, 0]
    loss_z = fin[3:4, 0]
    loss_total_hg = fin[4, 0]
    loss_seg_hg = fin[5, 0]
    loss_emb_hg = fin[6, 0]

    pred = head[:, 0:1]
    return (pred, loss_total, loss_total_2d, loss_offset, loss_z,
            hm, hm_inv, loss_total_hg, loss_seg_hg, loss_emb_hg)


# speedup vs baseline: 1.0409x; 1.0409x over previous
import jax
import jax.numpy as jnp
from jax.experimental import pallas as pl
from jax.experimental.pallas import tpu as pltpu

_POS_W = 10.0
_SMOOTH = 1.0
_M_VAR = 1.0
_M_DIST = 5.0
_K_INST = 4
_K_ROWS = 8
_L = 128
_TH = 64
_VMEM = 64 * 1024 * 1024


def _params(*sem):
    return pltpu.CompilerParams(dimension_semantics=tuple(sem),
                                vmem_limit_bytes=_VMEM)


def _sp_neg(x):
    return jnp.maximum(-x, 0.0) + jnp.log(1.0 + jnp.exp(-jnp.abs(x)))


def _k1_body(x_ref, w_ref, bias_ref, gseg_ref, goff_ref, gz_ref, iseg_ref,
             head_ref, b0_ref, i0_ref, t0_ref, ob_ref, zb_ref,
             b2_ref, i2_ref, t2_ref):
    @pl.when(pl.program_id(1) == 0)
    def _init():
        for r in (b0_ref, i0_ref, t0_ref, ob_ref, zb_ref, b2_ref, i2_ref, t2_ref):
            r[...] = jnp.zeros_like(r)

    w = w_ref[...]
    bias = bias_ref[...]
    r0, r3, r4, r5 = [], [], [], []
    for th in range(_TH):
        h = jnp.dot(w, x_ref[:, th, :],
                    preferred_element_type=jnp.float32) + bias
        head_ref[:, th, :] = h
        r0.append(h[0:1])
        r3.append(h[3:4])
        r4.append(h[4:5])
        r5.append(h[5:6])
    x0 = jnp.concatenate(r0, axis=0)
    x3 = jnp.concatenate(r3, axis=0)
    x4 = jnp.concatenate(r4, axis=0)
    x5 = jnp.concatenate(r5, axis=0)

    seg = gseg_ref[0]

    sp0 = _sp_neg(x0)
    sg0 = jnp.exp(-sp0)
    b0_ref[...] += _POS_W * seg * sp0 + (1.0 - seg) * (sp0 + x0)
    i0_ref[...] += sg0 * seg
    t0_ref[...] += sg0 + seg

    spn = _sp_neg(x3)
    fg = seg > 0.5
    nlp = jnp.where(fg, jnp.minimum(spn, 100.0), 100.0)
    nl1 = jnp.where(fg, jnp.minimum(spn + x3, 100.0), 0.0)
    t_off = goff_ref[0]
    ob_ref[...] += t_off * nlp + (1.0 - t_off) * nl1

    err = seg * x4 - gz_ref[0]
    zb_ref[...] += err * err

    t2 = iseg_ref[0]
    sp2 = _sp_neg(x5)
    sg2 = jnp.exp(-sp2)
    b2_ref[...] += _POS_W * t2 * sp2 + (1.0 - t2) * (sp2 + x5)
    i2_ref[...] += sg2 * t2
    t2_ref[...] += sg2 + t2


def _run_k1(x_bchw, w_t, bias, gseg, goff, gz, iseg):
    B, cin, H, W = x_bchw.shape
    nt = H // _TH
    gt_spec = pl.BlockSpec((None, 1, _TH, W), lambda b, t: (b, 0, t, 0))
    acc_spec = pl.BlockSpec((None, _TH, W), lambda b, t: (b, 0, 0))
    outs = pl.pallas_call(
        _k1_body,
        out_shape=(jax.ShapeDtypeStruct((B, 8, H, W), jnp.float32),)
        + (jax.ShapeDtypeStruct((B, _TH, W), jnp.float32),) * 8,
        grid=(B, nt),
        in_specs=[pl.BlockSpec((None, cin, _TH, W), lambda b, t: (b, 0, t, 0)),
                  pl.BlockSpec((8, cin), lambda b, t: (0, 0)),
                  pl.BlockSpec((8, 1), lambda b, t: (0, 0)),
                  gt_spec, gt_spec, gt_spec, gt_spec],
        out_specs=(pl.BlockSpec((None, 8, _TH, W), lambda b, t: (b, 0, t, 0)),)
        + (acc_spec,) * 8,
        compiler_params=_params("parallel", "arbitrary"),
    )(x_bchw, w_t, bias, gseg, goff, gz, iseg)
    return outs[0], outs[1:]


def _inst_masks(inst, P):
    ids_i = jax.lax.broadcasted_iota(jnp.int32, (_K_ROWS, P), 0) + 1
    onehot = jnp.where((inst == ids_i.astype(jnp.float32)) & (ids_i <= _K_INST),
                       1.0, 0.0)
    cnt = jnp.sum(onehot, axis=1, keepdims=True)
    valid = jnp.where(cnt > 0.0, 1.0, 0.0)
    safe = jnp.maximum(cnt, 1.0)
    ones_p = jnp.ones((1, P), jnp.float32)
    cnt_row = jax.lax.dot_general(ones_p, onehot, (((1,), (1,)), ((), ())),
                                  preferred_element_type=jnp.float32)
    valid_row = jnp.where(cnt_row > 0.0, 1.0, 0.0)
    return onehot, cnt, valid, safe, valid_row


def _pushpull(emb, masks):
    onehot, cnt, valid, safe, valid_row = masks
    sums = jax.lax.dot_general(onehot, emb, (((1,), (1,)), ((), ())),
                               preferred_element_type=jnp.float32)
    centers = sums / safe
    e_sq = jnp.sum(emb * emb, axis=0, keepdims=True)
    c_sq = jnp.sum(centers * centers, axis=1, keepdims=True)
    ec = jnp.dot(centers, emb, preferred_element_type=jnp.float32)
    dist = jnp.sqrt(jnp.maximum(e_sq - 2.0 * ec + c_sq, 0.0))
    pull = jnp.sum(valid * onehot * jnp.maximum(dist - _M_VAR, 0.0) / safe)

    cc = jax.lax.dot_general(centers, centers, (((1,), (1,)), ((), ())),
                             preferred_element_type=jnp.float32)
    ri = jax.lax.broadcasted_iota(jnp.int32, (_K_ROWS, _K_ROWS), 0)
    ci = jax.lax.broadcasted_iota(jnp.int32, (_K_ROWS, _K_ROWS), 1)
    eye = jnp.where(ri == ci, 1.0, 0.0)
    diag = jnp.sum(cc * eye, axis=0, keepdims=True)
    cd = jnp.sqrt(jnp.maximum(c_sq + diag - 2.0 * cc, 0.0))
    push = jnp.sum(valid * valid_row * (1.0 - eye)
                   * jnp.maximum(_M_DIST - cd, 0.0))
    return pull, push, jnp.sum(valid)


def _k2a_body(emb_ref, emb2_ref, inst_ref, inst2_ref, *out_refs):
    P = emb_ref.shape[1]
    row = jax.lax.broadcasted_iota(jnp.int32, (_K_ROWS, P), 0)
    ids = (row % 4 + 1).astype(jnp.float32)
    lo = row < 4
    inst_cat = jnp.where(lo, inst_ref[...], inst2_ref[...])
    onehot = jnp.where(inst_cat == ids, 1.0, 0.0)
    cnt = jnp.sum(onehot, axis=1, keepdims=True)
    valid = jnp.where(cnt > 0.0, 1.0, 0.0)
    safe = jnp.maximum(cnt, 1.0)
    ones_p = jnp.ones((1, P), jnp.float32)
    cnt_row = jax.lax.dot_general(ones_p, onehot, (((1,), (1,)), ((), ())),
                                  preferred_element_type=jnp.float32)
    valid_row = jnp.where(cnt_row > 0.0, 1.0, 0.0)

    emb = emb_ref[...]
    emb2 = emb2_ref[...]
    ecat = jnp.concatenate([emb, emb2], axis=0)
    sums = jax.lax.dot_general(onehot, ecat, (((1,), (1,)), ((), ())),
                               preferred_element_type=jnp.float32)
    ri4 = jax.lax.broadcasted_iota(jnp.int32, (_K_ROWS, 4), 0)
    ci4 = jax.lax.broadcasted_iota(jnp.int32, (_K_ROWS, 4), 1)
    blockm = jnp.where((ri4 < 4) == (ci4 < 2), 1.0, 0.0)
    centers = (sums / safe) * blockm
    ec = jnp.dot(centers, ecat, preferred_element_type=jnp.float32)
    e1 = jnp.sum(emb * emb, axis=0, keepdims=True)
    e2 = jnp.sum(emb2 * emb2, axis=0, keepdims=True)
    e_cat = jnp.where(lo, e1, e2)
    c_sq = jnp.sum(centers * centers, axis=1, keepdims=True)
    dist = jnp.sqrt(jnp.maximum(e_cat - 2.0 * ec + c_sq, 0.0))
    pull_t = valid * onehot * jnp.maximum(dist - _M_VAR, 0.0) / safe
    pu1 = jnp.sum(pull_t[0:4])
    pu2 = jnp.sum(pull_t[4:8])

    cc = jax.lax.dot_general(centers, centers, (((1,), (1,)), ((), ())),
                             preferred_element_type=jnp.float32)
    ri = jax.lax.broadcasted_iota(jnp.int32, (_K_ROWS, _K_ROWS), 0)
    ci = jax.lax.broadcasted_iota(jnp.int32, (_K_ROWS, _K_ROWS), 1)
    eye = jnp.where(ri == ci, 1.0, 0.0)
    same_branch = jnp.where((ri < 4) == (ci < 4), 1.0, 0.0)
    diag = jnp.sum(cc * eye, axis=0, keepdims=True)
    cd = jnp.sqrt(jnp.maximum(c_sq + diag - 2.0 * cc, 0.0))
    push_t = (valid * valid_row * (1.0 - eye) * same_branch
              * jnp.maximum(_M_DIST - cd, 0.0))
    ps1 = jnp.sum(push_t[0:4])
    ps2 = jnp.sum(push_t[4:8])
    c1 = jnp.sum(valid[0:4])
    c2 = jnp.sum(valid[4:8])

    for r, s in zip(out_refs, (pu1, ps1, c1, pu2, ps2, c2)):
        r[...] = jnp.broadcast_to(s, (1, _L))


def _run_k2a(emb, emb2, inst, inst2):
    B, _, P = emb.shape
    hb = B // 2
    bidx = lambda i, j: i * hb + j
    e_spec = pl.BlockSpec((None, 2, P), lambda i, j: (bidx(i, j), 0, 0))
    i_spec = pl.BlockSpec((None, 1, P), lambda i, j: (bidx(i, j), 0, 0))
    o_spec = pl.BlockSpec((None, 1, _L), lambda i, j: (bidx(i, j), 0, 0))
    return pl.pallas_call(
        _k2a_body,
        out_shape=(jax.ShapeDtypeStruct((B, 1, _L), jnp.float32),) * 6,
        grid=(2, hb),
        in_specs=[e_spec, e_spec, i_spec, i_spec],
        out_specs=(o_spec,) * 6,
        compiler_params=_params("parallel", "arbitrary"),
    )(emb, emb2, inst, inst2)


def _k2b_body(wemb_ref, inst2_ref, predw_ref, isegr_ref, *out_refs):
    P = wemb_ref.shape[1]
    pu3, ps3, c3 = _pushpull(wemb_ref[...], _inst_masks(inst2_ref[...], P))

    x = predw_ref[...]
    t = isegr_ref[...]
    sp = _sp_neg(x)
    sg = jnp.exp(-sp)
    bce = jnp.sum(_POS_W * t * sp + (1.0 - t) * (sp + x), axis=0, keepdims=True)
    inter = jnp.sum(sg * t, axis=0, keepdims=True)
    total = jnp.sum(sg + t, axis=0, keepdims=True)

    for r, s in zip(out_refs[:3], (pu3, ps3, c3)):
        r[...] = jnp.broadcast_to(s, (1, _L))
    out_refs[3][...] = bce
    out_refs[4][...] = inter
    out_refs[5][...] = total


def _run_k2b(wemb, inst2, predw, isegr):
    B, _, P = wemb.shape
    R = predw.shape[1]
    hb = B // 2
    bidx = lambda i, j: i * hb + j
    e_spec = pl.BlockSpec((None, 2, P), lambda i, j: (bidx(i, j), 0, 0))
    i_spec = pl.BlockSpec((None, 1, P), lambda i, j: (bidx(i, j), 0, 0))
    r_spec = pl.BlockSpec((None, R, _L), lambda i, j: (bidx(i, j), 0, 0))
    o_spec = pl.BlockSpec((None, 1, _L), lambda i, j: (bidx(i, j), 0, 0))
    return pl.pallas_call(
        _k2b_body,
        out_shape=(jax.ShapeDtypeStruct((B, 1, _L), jnp.float32),) * 6,
        grid=(2, hb),
        in_specs=[e_spec, i_spec, r_spec, r_spec],
        out_specs=(o_spec,) * 6,
        compiler_params=_params("parallel", "arbitrary"),
    )(wemb, inst2, predw, isegr)


def _make_k3_body(n):
    def _k3_body(b0_ref, i0_ref, t0_ref, ob_ref, zb_ref, b2_ref, i2_ref,
                 t2_ref, pp_a_ref, pp_b_ref, seg_h_ref, out_ref):
        _k3_math(n, b0_ref, i0_ref, t0_ref, ob_ref, zb_ref, b2_ref, i2_ref,
                 t2_ref, pp_a_ref, pp_b_ref, seg_h_ref, out_ref)
    return _k3_body


def _k3_math(n, b0_ref, i0_ref, t0_ref, ob_ref, zb_ref, b2_ref, i2_ref,
             t2_ref, pp_a_ref, pp_b_ref, seg_h_ref, out_ref):
    def _seg(bce, inter, total):
        i_s = jnp.sum(inter)
        u_s = jnp.sum(total) - i_s
        return bce / n + (1.0 - (i_s + _SMOOTH) / (u_s + _SMOOTH))

    loss_seg = _seg(jnp.sum(b0_ref[...]), i0_ref[...], t0_ref[...])
    loss_seg_2d = _seg(jnp.sum(b2_ref[...]), i2_ref[...], t2_ref[...])
    loss_offset_m = jnp.sum(ob_ref[...]) / n
    loss_z_m = jnp.sum(zb_ref[...]) / n

    ppa = jnp.sum(pp_a_ref[...], axis=(0, 2)) * (1.0 / _L)
    ppb = jnp.sum(pp_b_ref[...], axis=(0, 2)) * (1.0 / _L)

    def _pp_total(pu, ps, c):
        return (ps + pu) / jnp.maximum(c, 1.0)

    loss_emb = _pp_total(ppa[0], ppa[1], ppa[2])
    loss_emb_2d = _pp_total(ppa[3], ppa[4], ppa[5])
    loss_emb_hg = _pp_total(ppb[0], ppb[1], ppb[2])

    sh = seg_h_ref[...]
    loss_seg_hg = _seg(jnp.sum(sh[:, 0]), sh[:, 1], sh[:, 2])

    vals = (3.0 * loss_seg + 0.5 * loss_emb,
            3.0 * loss_seg_2d + 0.5 * loss_emb_2d,
            60.0 * loss_offset_m,
            30.0 * loss_z_m,
            3.0 * loss_seg_hg + 0.5 * loss_emb_hg,
            loss_seg_hg,
            loss_emb_hg,
            loss_seg_hg)
    out_ref[...] = jnp.concatenate(
        [jnp.broadcast_to(v, (1, _L)) for v in vals], axis=0)


def _run_k3(accs, ppa, ppb, seg_h, n):
    B = ppa.shape[0]
    full = lambda s: pl.BlockSpec(s, lambda: tuple(0 for _ in s))
    in_specs = [full(a.shape) for a in accs] + [
        full((B, 6, _L)), full((B, 3, _L)), full((B, 3, _L))]
    return pl.pallas_call(
        _make_k3_body(n),
        out_shape=jax.ShapeDtypeStruct((8, _L), jnp.float32),
        in_specs=in_specs,
        out_specs=full((8, _L)),
        compiler_params=_params(),
    )(*accs, ppa, ppb, seg_h)


def _inv3(m):
    a, b, c = m[..., 0, 0], m[..., 0, 1], m[..., 0, 2]
    d, e, f = m[..., 1, 0], m[..., 1, 1], m[..., 1, 2]
    g, h, i = m[..., 2, 0], m[..., 2, 1], m[..., 2, 2]
    ca = e * i - f * h
    cb = -(d * i - f * g)
    cc = d * h - e * g
    cd = -(b * i - c * h)
    ce = a * i - c * g
    cf = -(a * h - b * g)
    cg = b * f - c * e
    ch = -(a * f - c * d)
    ci = a * e - b * d
    det = a * ca + b * cb + c * cc
    adj = jnp.stack([jnp.stack([ca, cd, cg], -1),
                     jnp.stack([cb, ce, ch], -1),
                     jnp.stack([cc, cf, ci], -1)], -2)
    return adj / det[..., None, None]


def _norm_pix(h, w):
    wd = float(w - 1) if w != 1 else 1e-14
    hd = float(h - 1) if h != 1 else 1e-14
    return jnp.array([[2.0 / wd, 0.0, -1.0],
                      [0.0, 2.0 / hd, -1.0],
                      [0.0, 0.0, 1.0]], jnp.float32)


def _warp3(src_flat, m_pix, H, W):
    n_t = _norm_pix(H, W)
    m_norm = n_t @ (m_pix @ _inv3(n_t))
    m_s2d = _inv3(m_norm)
    xs = jnp.linspace(-1.0, 1.0, W, dtype=jnp.float32)
    ys = jnp.linspace(-1.0, 1.0, H, dtype=jnp.float32)
    gx, gy = jnp.meshgrid(xs, ys)
    pts = jnp.stack([gx.ravel(), gy.ravel(),
                     jnp.ones(H * W, jnp.float32)], -1)
    tp = pts @ m_s2d.T
    zd = tp[:, 2]
    zd = jnp.where(jnp.abs(zd) > 1e-8, zd, 1e-8)
    sx = (tp[:, 0] / zd + 1.0) * 0.5 * (W - 1)
    sy = (tp[:, 1] / zd + 1.0) * 0.5 * (H - 1)
    x0 = jnp.floor(sx)
    y0 = jnp.floor(sy)
    wx1 = sx - x0
    wy1 = sy - y0

    def tap(ix, iy, wgt):
        ok = ((ix >= 0) & (ix <= W - 1) & (iy >= 0) & (iy <= H - 1))
        idx = (jnp.clip(iy, 0, H - 1) * W + jnp.clip(ix, 0, W - 1)).astype(jnp.int32)
        g = jnp.take(src_flat, idx, axis=1)
        return g * jnp.where(ok, wgt, 0.0)[None, :]

    return (tap(x0, y0, (1.0 - wx1) * (1.0 - wy1))
            + tap(x0 + 1.0, y0, wx1 * (1.0 - wy1))
            + tap(x0, y0 + 1.0, (1.0 - wx1) * wy1)
            + tap(x0 + 1.0, y0 + 1.0, wx1 * wy1))


def kernel(inputs, images_gt, gt_seg, gt_instance, gt_offset_y, gt_z,
           image_gt_segment, image_gt_instance, w_bev, b_bev, w_2d, b_2d,
           homography):
    B, cin, H, W = inputs.shape
    P = H * W
    R = P // _L
    w_t = jnp.transpose(jnp.concatenate([w_bev, w_2d], axis=1))
    bias = jnp.transpose(jnp.concatenate([b_bev, b_2d], axis=1))

    head, accs = _run_k1(inputs, w_t, bias, gt_seg, gt_offset_y, gt_z,
                         image_gt_segment)

    hm = jnp.broadcast_to(homography[None], (B, 3, 3))
    hm_inv = _inv3(hm)
    fro = jnp.sqrt(jnp.sum(hm_inv * hm_inv, axis=(1, 2), keepdims=True))
    hm_inv = hm_inv / jnp.maximum(fro, 1e-6)
    n_t = _norm_pix(H, W)[None]
    hm_inv = _inv3(n_t) @ (hm_inv @ n_t)

    emb = head[:, 1:3].reshape(B, 2, P)
    emb2 = head[:, 6:8].reshape(B, 2, P)
    inst = gt_instance.reshape(B, 1, P)
    inst2 = image_gt_instance.reshape(B, 1, P)
    k2a = _run_k2a(emb, emb2, inst, inst2)

    warp_src = head[:, 5:8].reshape(B * 3, P)
    warped = _warp3(warp_src, hm_inv[0], H, W).reshape(B, 3, P)
    wemb = warped[:, 1:3]
    predw = warped[:, 0].reshape(B, R, _L)
    isegr = image_gt_segment.reshape(B, R, _L)
    k2b = _run_k2b(wemb, inst2, predw, isegr)

    ppa = jnp.concatenate(k2a, axis=1)
    ppb = jnp.concatenate(k2b[:3], axis=1)
    seg_h = jnp.concatenate(k2b[3:], axis=1)
    fin = _run_k3(accs, ppa, ppb, seg_h, float(B * P))

    loss_total = fin[0:1, 0]
    loss_total_2d = fin[1:2, 0]
    loss_offset = fin[2:3, 0]
    loss_z = fin[3:4, 0]
    loss_total_hg = fin[4, 0]
    loss_seg_hg = fin[5, 0]
    loss_emb_hg = fin[6, 0]

    pred = head[:, 0:1]
    return (pred, loss_total, loss_total_2d, loss_offset, loss_z,
            hm, hm_inv, loss_total_hg, loss_seg_hg, loss_emb_hg)
